# baseline (speedup 1.0000x reference)
import functools

import jax
import jax.numpy as jnp
from jax import lax
from jax.experimental import pallas as pl
from jax.experimental.pallas import tpu as pltpu
from jax.experimental.pallas import tpu_sc as plsc

EPS_LN = 1e-12

_W = 128


def _sc_gather(table, idx2d, bl, h):
    mesh = plsc.VectorSubcoreMesh(core_axis_name="c", subcore_axis_name="s")

    @functools.partial(
        pl.kernel,
        out_type=jax.ShapeDtypeStruct((bl, h), jnp.float32),
        mesh=mesh,
        compiler_params=pltpu.CompilerParams(use_tc_tiling_on_sc=False),
    )
    def gather_kernel(tbl_hbm, idx_hbm, out_hbm):
        def body(i_vmem, o_vmem):
            pltpu.sync_copy(tbl_hbm.at[i_vmem.at[0]], o_vmem)

        pltpu.emit_pipeline(
            body,
            grid=(bl // _W,),
            in_specs=[pl.BlockSpec((1, _W), lambda i: (0, i))],
            out_specs=[pl.BlockSpec((_W, h), lambda i: (i, 0))],
            core_axis_name=("c", "s"),
            dimension_semantics=(pltpu.PARALLEL,),
        )(idx_hbm, out_hbm)

    return gather_kernel(table, idx2d)


def _ln_body(x_ref, p_ref, g_ref, b_ref, o_ref):
    x = x_ref[...] + p_ref[...]
    mu = jnp.mean(x, axis=-1, keepdims=True)
    xc = x - mu
    var = jnp.mean(xc * xc, axis=-1, keepdims=True)
    o_ref[...] = xc * lax.rsqrt(var + EPS_LN) * g_ref[...] + b_ref[...]


def _tc_ln(gathered, pos, gamma, beta, bb):
    b, l, h = gathered.shape
    return pl.pallas_call(
        _ln_body,
        grid=(b // bb,),
        in_specs=[
            pl.BlockSpec((bb, l, h), lambda i: (i, 0, 0)),
            pl.BlockSpec((1, l, h), lambda i: (0, 0, 0)),
            pl.BlockSpec((1, 1, h), lambda i: (0, 0, 0)),
            pl.BlockSpec((1, 1, h), lambda i: (0, 0, 0)),
        ],
        out_specs=pl.BlockSpec((bb, l, h), lambda i: (i, 0, 0)),
        out_shape=jax.ShapeDtypeStruct((b, l, h), jnp.float32),
    )(gathered, pos, gamma, beta)


def kernel(prefix_text, word_embeddings, position_embeddings, ln_gamma, ln_beta):
    b, l = prefix_text.shape
    v, h = word_embeddings.shape
    bl = b * l
    idx2d = prefix_text.astype(jnp.int32).reshape(1, bl)
    gathered = _sc_gather(word_embeddings, idx2d, bl, h)
    pos = position_embeddings[:l].reshape(1, l, h)
    gamma = ln_gamma.reshape(1, 1, h)
    beta = ln_beta.reshape(1, 1, h)
    return _tc_ln(gathered.reshape(b, l, h), pos, gamma, beta, bb=64)
